# Initial kernel scaffold; baseline (speedup 1.0000x reference)
import jax
import jax.numpy as jnp
from jax.experimental import pallas as pl


def kernel(x, A, B):
    raise NotImplementedError("write your pallas kernel here")


# speedup vs baseline: 1.4795x; 1.4795x over previous
import functools

import jax
import jax.numpy as jnp
from jax.experimental import pallas as pl
from jax.experimental.pallas import tpu as pltpu

N_F = 1024
N_K = 4096
B_TILE = 256


def _prep_kernel(a_ref, an_ref):
    a = a_ref[...]
    inv = jax.lax.rsqrt(jnp.sum(a * a, axis=0, keepdims=True))
    an_ref[...] = (a * inv).astype(jnp.bfloat16)


def _fused_kernel(x_ref, an_ref, b_ref, inner_ref, out_ref):
    xb = x_ref[...].astype(jnp.bfloat16)
    inner = jnp.dot(xb, an_ref[...], preferred_element_type=jnp.float32)
    inner_ref[...] = inner
    out_ref[...] = jnp.dot(inner.astype(jnp.bfloat16), b_ref[...],
                           preferred_element_type=jnp.float32)


@functools.partial(jax.jit, static_argnums=())
def kernel(x, A, B):
    batch = x.shape[0]
    An = pl.pallas_call(
        _prep_kernel,
        out_shape=jax.ShapeDtypeStruct((N_F, N_K), jnp.bfloat16),
    )(A)
    Bb = B.astype(jnp.bfloat16)
    grid = (batch // B_TILE,)
    inner, out = pl.pallas_call(
        _fused_kernel,
        grid=grid,
        in_specs=[
            pl.BlockSpec((B_TILE, N_F), lambda i: (i, 0)),
            pl.BlockSpec((N_F, N_K), lambda i: (0, 0)),
            pl.BlockSpec((N_K, N_F), lambda i: (0, 0)),
        ],
        out_specs=[
            pl.BlockSpec((B_TILE, N_K), lambda i: (i, 0)),
            pl.BlockSpec((B_TILE, N_F), lambda i: (i, 0)),
        ],
        out_shape=[
            jax.ShapeDtypeStruct((batch, N_K), jnp.float32),
            jax.ShapeDtypeStruct((batch, N_F), jnp.float32),
        ],
        compiler_params=pltpu.CompilerParams(
            dimension_semantics=("parallel",),
        ),
    )(x, An, Bb)
    return (out, inner)


# speedup vs baseline: 1.5187x; 1.0265x over previous
import functools

import jax
import jax.numpy as jnp
from jax.experimental import pallas as pl
from jax.experimental.pallas import tpu as pltpu

N_F = 1024
N_K = 4096
B_TILE = 512


def _prep_kernel(a_ref, an_ref):
    a = a_ref[...]
    inv = jax.lax.rsqrt(jnp.sum(a * a, axis=0, keepdims=True))
    an_ref[...] = (a * inv).astype(jnp.bfloat16)


def _fused_kernel(x_ref, an_ref, b_ref, inner_ref, out_ref):
    xb = x_ref[...].astype(jnp.bfloat16)
    inner = jnp.dot(xb, an_ref[...], preferred_element_type=jnp.float32)
    inner_ref[...] = inner
    out_ref[...] = jnp.dot(inner.astype(jnp.bfloat16), b_ref[...],
                           preferred_element_type=jnp.float32)


@functools.partial(jax.jit, static_argnums=())
def kernel(x, A, B):
    batch = x.shape[0]
    An = pl.pallas_call(
        _prep_kernel,
        out_shape=jax.ShapeDtypeStruct((N_F, N_K), jnp.bfloat16),
    )(A)
    Bb = B.astype(jnp.bfloat16)
    grid = (batch // B_TILE,)
    inner, out = pl.pallas_call(
        _fused_kernel,
        grid=grid,
        in_specs=[
            pl.BlockSpec((B_TILE, N_F), lambda i: (i, 0)),
            pl.BlockSpec((N_F, N_K), lambda i: (0, 0)),
            pl.BlockSpec((N_K, N_F), lambda i: (0, 0)),
        ],
        out_specs=[
            pl.BlockSpec((B_TILE, N_K), lambda i: (i, 0)),
            pl.BlockSpec((B_TILE, N_F), lambda i: (i, 0)),
        ],
        out_shape=[
            jax.ShapeDtypeStruct((batch, N_K), jnp.float32),
            jax.ShapeDtypeStruct((batch, N_F), jnp.float32),
        ],
        compiler_params=pltpu.CompilerParams(
            dimension_semantics=("parallel",),
        ),
    )(x, An, Bb)
    return (out, inner)
